# baseline (speedup 1.0000x reference)
import functools

import numpy as np

import jax
import jax.numpy as jnp
from jax import lax
from jax.experimental import pallas as pl
from jax.experimental.pallas import tpu as pltpu
from jax.experimental.pallas import tpu_sc as plsc

F32 = jnp.float32
I32 = jnp.int32

_N = 10000
_E = 160000
_NP = 10240
_NPW = 640
_DUMMY = _NPW
_EPS = 1e-5
_INF = np.float32(3.0e38)

_EH = _E // 2
_CH = 8000
_NCHUNK = _EH // _CH
_FLUSH = _CH + 16
_CAPR = 88576

_mesh = plsc.VectorSubcoreMesh(core_axis_name="c", subcore_axis_name="s")


def _sc_gather_body(xp_hbm, src_hbm, xs_hbm, srcb0, srcb1, srcb16, rows0,
                    rows1, xsb0, xsb1, ss0, ss1, sg0, sg1, so0, so1):
    wid = lax.axis_index("s") * 2 + lax.axis_index("c")
    base = wid * (_E // 32)
    nb = 39

    srcbs = (srcb0, srcb1)
    rowss = (rows0, rows1)
    xsbs = (xsb0, xsb1)
    sss = (ss0, ss1)
    sgs = (sg0, sg1)
    sos = (so0, so1)

    def s_src(b, slot):
        pltpu.async_copy(src_hbm.at[pl.ds(base + b * 128, 128)], srcbs[slot], sss[slot])

    def w_src(slot):
        pltpu.make_async_copy(src_hbm.at[pl.ds(base, 128)], srcbs[slot], sss[slot]).wait()

    def s_g(slot):
        pltpu.async_copy(xp_hbm.at[srcbs[slot]], rowss[slot], sgs[slot])

    def w_g(slot):
        pltpu.make_async_copy(xp_hbm.at[srcbs[slot]], rowss[slot], sgs[slot]).wait()

    def s_out(b, slot):
        pltpu.async_copy(xsbs[slot], xs_hbm.at[pl.ds(base + b * 128, 128), :], sos[slot])

    def w_out(slot):
        pltpu.make_async_copy(xsbs[slot], xs_hbm.at[pl.ds(base, 128), :], sos[slot]).wait()

    s_src(0, 0)
    w_src(0)
    s_g(0)
    s_src(1, 1)

    def pairloop(g, _):
        for slot in (0, 1):
            b = g * 2 + slot

            @pl.when(b < nb)
            def _():
                @pl.when(b + 1 < nb)
                def _():
                    w_src((slot + 1) % 2)
                    s_g((slot + 1) % 2)

                w_g(slot)

                @pl.when(b + 2 < nb)
                def _():
                    s_src(b + 2, slot)

                @pl.when(b >= 2)
                def _():
                    w_out(slot)

                rows = rowss[slot]
                xsb = xsbs[slot]

                def crow(r, _):
                    xsb[r, :] = rows[r, pl.ds(0, 16)]
                    return 0

                lax.fori_loop(0, 128, crow, 0)
                s_out(b, slot)
        return 0

    lax.fori_loop(0, (nb + 1) // 2, pairloop, 0)
    w_out((nb - 2) % 2)
    w_out((nb - 1) % 2)

    srcb16[:] = jnp.zeros((16,), I32)
    e0 = base + 39 * 128
    pltpu.sync_copy(src_hbm.at[pl.ds(e0, 8)], srcb16.at[pl.ds(0, 8)])
    pltpu.async_copy(xp_hbm.at[srcb16], rows0.at[pl.ds(0, 16), :], sg0).wait()

    def crow8(r, _):
        xsb0[r, :] = rows0[r, pl.ds(0, 16)]
        return 0

    lax.fori_loop(0, 8, crow8, 0)
    pltpu.sync_copy(xsb0.at[pl.ds(0, 8), :], xs_hbm.at[pl.ds(e0, 8), :])


def _sc_gather(xp128, src):
    return pl.kernel(
        _sc_gather_body,
        out_type=jax.ShapeDtypeStruct((_E, 16), F32),
        mesh=_mesh,
        compiler_params=pltpu.CompilerParams(needs_layout_passes=False),
        scratch_types=[
            pltpu.VMEM((128,), I32),
            pltpu.VMEM((128,), I32),
            pltpu.VMEM((16,), I32),
            pltpu.VMEM((128, 128), F32),
            pltpu.VMEM((128, 128), F32),
            pltpu.VMEM((128, 16), F32),
            pltpu.VMEM((128, 16), F32),
            pltpu.SemaphoreType.DMA,
            pltpu.SemaphoreType.DMA,
            pltpu.SemaphoreType.DMA,
            pltpu.SemaphoreType.DMA,
            pltpu.SemaphoreType.DMA,
            pltpu.SemaphoreType.DMA,
        ],
    )(xp128, src)


def _sc_segsum_body(msg0_hbm, msg1_hbm, dst_hbm, sum0_hbm, sum1_hbm,
                    deg0_hbm, deg1_hbm, idxb0, idxb1, idxb2, idxb40,
                    rows0, rows1, rows2, ones, zbuf, S,
                    sd0, sd1, sd2, sm0, sm1, sm2, sc0, sc1, sc2):
    cid = lax.axis_index("c")
    sid = lax.axis_index("s")

    idxbs = (idxb0, idxb1, idxb2)
    rowss = (rows0, rows1, rows2)
    sds = (sd0, sd1, sd2)
    sms = (sm0, sm1, sm2)
    scs = (sc0, sc1, sc2)

    z16 = jnp.zeros((16,), F32)
    one16 = jnp.ones((16,), F32)

    def zrow(i, _):
        for j in range(8):
            zbuf[i, pl.ds(j * 16, 16)] = z16
        return 0

    lax.fori_loop(0, 32, zrow, 0)

    def orow(i, _):
        for j in range(8):
            ones[i, pl.ds(j * 16, 16)] = one16
        return 0

    lax.fori_loop(0, 80, orow, 0)

    def zs(k, _):
        pltpu.sync_copy(zbuf, S.at[pl.ds(sid * _NPW + k * 32, 32), :])
        return 0

    lax.fori_loop(0, _NPW // 32, zs, 0)

    plsc.subcore_barrier()

    base = sid * (_E // 16)
    nb = 125

    def s_ld(b, slot):
        e0 = base + b * 80
        pltpu.async_copy(dst_hbm.at[pl.ds(e0, 80)], idxbs[slot], sds[slot])

        @pl.when(cid == 0)
        def _():
            pltpu.async_copy(msg0_hbm.at[pl.ds(e0, 80), :], rowss[slot], sms[slot])

        @pl.when(cid == 1)
        def _():
            pltpu.async_copy(msg1_hbm.at[pl.ds(e0, 80), :], rowss[slot], sms[slot])

    def w_ld(slot):
        pltpu.make_async_copy(dst_hbm.at[pl.ds(base, 80)], idxbs[slot], sds[slot]).wait()
        pltpu.make_async_copy(msg0_hbm.at[pl.ds(base, 80), :], rowss[slot], sms[slot]).wait()

    def s_sc(slot):
        pltpu.async_copy(rowss[slot], S.at[idxbs[slot]], scs[slot], add=True)

    def w_sc(slot):
        pltpu.make_async_copy(rowss[slot], S.at[idxbs[slot]], scs[slot]).wait()

    s_ld(0, 0)
    s_ld(1, 1)

    def triloop(g, _):
        for slot in (0, 1, 2):
            b = g * 3 + slot

            @pl.when(b < nb)
            def _():
                w_ld(slot)
                s_sc(slot)
                prev = (slot + 2) % 3

                @pl.when(b >= 1)
                def _():
                    w_sc(prev)

                @pl.when(b + 2 < nb)
                def _():
                    s_ld(b + 2, prev)
        return 0

    lax.fori_loop(0, (nb + 2) // 3, triloop, 0)
    w_sc((nb - 1) % 3)

    plsc.subcore_barrier()

    r0 = sid * _NPW

    @pl.when(cid == 0)
    def _():
        pltpu.sync_copy(S.at[pl.ds(r0, _NPW), :], sum0_hbm.at[pl.ds(r0, _NPW), :])

    @pl.when(cid == 1)
    def _():
        pltpu.sync_copy(S.at[pl.ds(r0, _NPW), :], sum1_hbm.at[pl.ds(r0, _NPW), :])

    plsc.subcore_barrier()

    def zs2(k, _):
        pltpu.sync_copy(zbuf, S.at[pl.ds(sid * _NPW + k * 32, 32), :])
        return 0

    lax.fori_loop(0, _NPW // 32, zs2, 0)
    plsc.subcore_barrier()

    dbase = cid * _EH + sid * (_EH // 16)
    nd = 62

    def d_ld(b, slot):
        pltpu.async_copy(dst_hbm.at[pl.ds(dbase + b * 80, 80)], idxbs[slot], sds[slot])

    def dw_ld(slot):
        pltpu.make_async_copy(dst_hbm.at[pl.ds(dbase, 80)], idxbs[slot], sds[slot]).wait()

    def d_sc(slot):
        pltpu.async_copy(ones, S.at[idxbs[slot]], scs[slot], add=True)

    def dw_sc(slot):
        pltpu.make_async_copy(ones, S.at[idxbs[slot]], scs[slot]).wait()

    d_ld(0, 0)
    d_ld(1, 1)

    def dtriloop(g, _):
        for slot in (0, 1, 2):
            b = g * 3 + slot

            @pl.when(b < nd)
            def _():
                dw_ld(slot)
                d_sc(slot)
                prev = (slot + 2) % 3

                @pl.when(b >= 1)
                def _():
                    dw_sc(prev)

                @pl.when(b + 2 < nd)
                def _():
                    d_ld(b + 2, prev)
        return 0

    lax.fori_loop(0, (nd + 2) // 3, dtriloop, 0)
    dw_sc((nd - 1) % 3)

    pltpu.sync_copy(dst_hbm.at[pl.ds(dbase + 4960, 40)], idxb40)
    pltpu.sync_copy(ones.at[pl.ds(0, 40), :], S.at[idxb40], add=True)

    plsc.subcore_barrier()

    @pl.when(cid == 0)
    def _():
        pltpu.sync_copy(S.at[pl.ds(r0, _NPW), :], deg0_hbm.at[pl.ds(r0, _NPW), :])

    @pl.when(cid == 1)
    def _():
        pltpu.sync_copy(S.at[pl.ds(r0, _NPW), :], deg1_hbm.at[pl.ds(r0, _NPW), :])


def _sc_segsum(msg0, msg1, dst):
    return pl.kernel(
        _sc_segsum_body,
        out_type=[
            jax.ShapeDtypeStruct((_NP, 128), F32),
            jax.ShapeDtypeStruct((_NP, 128), F32),
            jax.ShapeDtypeStruct((_NP, 128), F32),
            jax.ShapeDtypeStruct((_NP, 128), F32),
        ],
        mesh=_mesh,
        compiler_params=pltpu.CompilerParams(needs_layout_passes=False),
        scratch_types=[
            pltpu.VMEM((80,), I32),
            pltpu.VMEM((80,), I32),
            pltpu.VMEM((80,), I32),
            pltpu.VMEM((40,), I32),
            pltpu.VMEM((80, 128), F32),
            pltpu.VMEM((80, 128), F32),
            pltpu.VMEM((80, 128), F32),
            pltpu.VMEM((80, 128), F32),
            pltpu.VMEM((32, 128), F32),
            pltpu.VMEM_SHARED((_NP, 128), F32),
            pltpu.SemaphoreType.DMA,
            pltpu.SemaphoreType.DMA,
            pltpu.SemaphoreType.DMA,
            pltpu.SemaphoreType.DMA,
            pltpu.SemaphoreType.DMA,
            pltpu.SemaphoreType.DMA,
            pltpu.SemaphoreType.DMA,
            pltpu.SemaphoreType.DMA,
            pltpu.SemaphoreType.DMA,
        ],
    )(msg0, msg1, dst)


def _sc_part_body(src_hbm, dst_hbm, gsrc_hbm, gld_hbm, meta_hbm,
                  srcv0, srcv1, dstv0, dstv1, bufS0, bufS1, bufL0, bufL1,
                  dumS, dumL, metab, ss0, ss1, sd0, sd1, fS0, fS1, fL0, fL1):
    cid = lax.axis_index("c")
    sid = lax.axis_index("s")
    row = cid * 16 + sid
    ebase = cid * _EH
    lo = sid * _NPW

    srcvs = (srcv0, srcv1)
    dstvs = (dstv0, dstv1)
    bufSs = (bufS0, bufS1)
    bufLs = (bufL0, bufL1)
    sss = (ss0, ss1)
    sds = (sd0, sd1)
    fSs = (fS0, fS1)
    fLs = (fL0, fL1)

    z16 = jnp.zeros((16,), I32)
    d16 = jnp.full((16,), _DUMMY, I32)

    def fillbuf(i, _):
        bufS0[pl.ds(i * 16, 16)] = z16
        bufS1[pl.ds(i * 16, 16)] = z16
        bufL0[pl.ds(i * 16, 16)] = d16
        bufL1[pl.ds(i * 16, 16)] = d16
        return 0

    lax.fori_loop(0, _FLUSH // 16, fillbuf, 0)

    def filldum(i, _):
        dumS[pl.ds(i * 16, 16)] = z16
        dumL[pl.ds(i * 16, 16)] = d16
        return 0

    lax.fori_loop(0, 8, filldum, 0)

    def s_in(c, slot):
        pltpu.async_copy(src_hbm.at[pl.ds(ebase + c * _CH, _CH)], srcvs[slot], sss[slot])
        pltpu.async_copy(dst_hbm.at[pl.ds(ebase + c * _CH, _CH)], dstvs[slot], sds[slot])

    def w_in(slot):
        pltpu.make_async_copy(src_hbm.at[pl.ds(ebase, _CH)], srcvs[slot], sss[slot]).wait()
        pltpu.make_async_copy(dst_hbm.at[pl.ds(ebase, _CH)], dstvs[slot], sds[slot]).wait()

    def s_fl(slot, written):
        wo = pl.multiple_of(row * _CAPR + written, 16)
        pltpu.async_copy(bufSs[slot], gsrc_hbm.at[pl.ds(wo, _FLUSH)], fSs[slot])
        pltpu.async_copy(bufLs[slot], gld_hbm.at[pl.ds(wo, _FLUSH)], fLs[slot])

    def w_fl(slot):
        pltpu.make_async_copy(bufSs[slot], gsrc_hbm.at[pl.ds(0, _FLUSH)], fSs[slot]).wait()
        pltpu.make_async_copy(bufLs[slot], gld_hbm.at[pl.ds(0, _FLUSH)], fLs[slot]).wait()

    s_in(0, 0)
    s_in(1, 1)

    def pair(g, written):
        for slot in (0, 1):
            c = g * 2 + slot
            w_in(slot)

            @pl.when(c >= 2)
            def _():
                w_fl(slot)

            srcv = srcvs[slot]
            dstv = dstvs[slot]
            bufS = bufSs[slot]
            bufL = bufLs[slot]

            def scan(i, off):
                d = dstv[pl.ds(i * 16, 16)]
                sv = srcv[pl.ds(i * 16, 16)]
                m = (d >= lo) & (d < lo + _NPW)
                plsc.store_compressed(bufS.at[pl.ds(off, 16)], sv, mask=m)
                plsc.store_compressed(bufL.at[pl.ds(off, 16)], d - lo, mask=m)
                return off + jnp.sum(m.astype(I32))

            off = lax.fori_loop(0, _CH // 16, scan, 0)
            bufS[pl.ds(off, 16)] = z16
            bufL[pl.ds(off, 16)] = d16
            off16 = (off + 15) & ~15
            s_fl(slot, written)

            @pl.when(c + 2 < _NCHUNK)
            def _():
                s_in(c + 2, slot)

            written = written + off16
        return written

    written = lax.fori_loop(0, _NCHUNK // 2, pair, 0)
    w_fl(0)
    w_fl(1)
    wo = pl.multiple_of(row * _CAPR + written, 16)
    pltpu.sync_copy(dumS, gsrc_hbm.at[pl.ds(wo, 128)])
    pltpu.sync_copy(dumL, gld_hbm.at[pl.ds(wo, 128)])
    m128 = (written + 127) & ~127
    metab[:] = jnp.zeros((16,), I32) + m128
    pltpu.sync_copy(metab, meta_hbm.at[pl.ds(pl.multiple_of(row * 16, 16), 16)])


def _sc_partition(src, dst):
    return pl.kernel(
        _sc_part_body,
        out_type=[
            jax.ShapeDtypeStruct((32 * _CAPR,), I32),
            jax.ShapeDtypeStruct((32 * _CAPR,), I32),
            jax.ShapeDtypeStruct((512,), I32),
        ],
        mesh=_mesh,
        compiler_params=pltpu.CompilerParams(needs_layout_passes=False),
        scratch_types=[
            pltpu.VMEM((_CH,), I32),
            pltpu.VMEM((_CH,), I32),
            pltpu.VMEM((_CH,), I32),
            pltpu.VMEM((_CH,), I32),
            pltpu.VMEM((_FLUSH,), I32),
            pltpu.VMEM((_FLUSH,), I32),
            pltpu.VMEM((_FLUSH,), I32),
            pltpu.VMEM((_FLUSH,), I32),
            pltpu.VMEM((128,), I32),
            pltpu.VMEM((128,), I32),
            pltpu.VMEM((16,), I32),
            pltpu.SemaphoreType.DMA,
            pltpu.SemaphoreType.DMA,
            pltpu.SemaphoreType.DMA,
            pltpu.SemaphoreType.DMA,
            pltpu.SemaphoreType.DMA,
            pltpu.SemaphoreType.DMA,
            pltpu.SemaphoreType.DMA,
            pltpu.SemaphoreType.DMA,
        ],
    )(src, dst)


def _sc_segmin_body(dc, nacc, b_hbm, gsrc_hbm, gld_hbm, meta_hbm, out0_hbm,
                    out1_hbm, idxb0, idxb1, ldv0, ldv1, rows0, rows1, metav,
                    *accs_and_sems):
    accs = accs_and_sems[:nacc]
    si0, si1, sl0, sl1, sg0, sg1 = accs_and_sems[-6:]
    acc = accs[0]
    cid = lax.axis_index("c")
    sid = lax.axis_index("s")
    row = cid * 16 + sid
    nj = dc // 16
    pf = 128 // dc
    shift = {1: 0, 2: 1, 4: 2}[pf]
    lmask = pf - 1

    idxbs = (idxb0, idxb1)
    ldvs = (ldv0, ldv1)
    rowss = (rows0, rows1)
    sis = (si0, si1)
    sls = (sl0, sl1)
    sgs = (sg0, sg1)

    inf16 = jnp.full((16,), _INF, F32)

    def initrow(i, _):
        for a in accs:
            for j in range(8):
                a[i, pl.ds(j * 16, 16)] = inf16
        return 0

    lax.fori_loop(0, _NPW // pf + 1, initrow, 0)

    pltpu.sync_copy(meta_hbm, metav)
    m128 = jnp.max(metav[pl.ds(pl.multiple_of(row * 16, 16), 16)])
    nb = m128 // 128

    def s_idx(b, slot):
        bo = pl.multiple_of(row * _CAPR + b * 128, 128)
        pltpu.async_copy(gsrc_hbm.at[pl.ds(bo, 128)], idxbs[slot], sis[slot])
        pltpu.async_copy(gld_hbm.at[pl.ds(bo, 128)], ldvs[slot], sls[slot])

    def w_idx(slot):
        pltpu.make_async_copy(gsrc_hbm.at[pl.ds(0, 128)], idxbs[slot], sis[slot]).wait()
        pltpu.make_async_copy(gld_hbm.at[pl.ds(0, 128)], ldvs[slot], sls[slot]).wait()

    def s_g(slot):
        pltpu.async_copy(b_hbm.at[idxbs[slot]], rowss[slot], sgs[slot])

    def w_g(slot):
        pltpu.make_async_copy(b_hbm.at[idxbs[slot]], rowss[slot], sgs[slot]).wait()

    def upd(slot):
        ldv = ldvs[slot]
        rows = rowss[slot]

        def qloop(q, _):
            ld16 = ldv[pl.ds(q * 16, 16)]
            for i in range(16):
                ld = ld16[i]
                t = q * 16 + i
                a_t = accs[i % nacc]
                if pf == 1:
                    rr = ld
                    cb = 0
                else:
                    rr = ld >> shift
                    cb = (ld & lmask) * dc
                for j in range(nj):
                    a = a_t[rr, pl.ds(cb + j * 16, 16)]
                    r = rows[t, pl.ds(j * 16, 16)]
                    a_t[rr, pl.ds(cb + j * 16, 16)] = jnp.minimum(a, r)
            return 0

        lax.fori_loop(0, 8, qloop, 0)

    @pl.when(nb > 0)
    def _():
        s_idx(0, 0)
        w_idx(0)
        s_g(0)

        @pl.when(nb > 1)
        def _():
            s_idx(1, 1)

    def pairloop(g, _):
        for slot in (0, 1):
            b = g * 2 + slot

            @pl.when(b < nb)
            def _():
                @pl.when(b + 1 < nb)
                def _():
                    w_idx((slot + 1) % 2)
                    s_g((slot + 1) % 2)

                w_g(slot)
                upd(slot)

                @pl.when(b + 2 < nb)
                def _():
                    s_idx(b + 2, slot)
        return 0

    lax.fori_loop(0, (nb + 1) // 2, pairloop, 0)

    r0 = sid * _NPW

    if nacc == 1:
        @pl.when(cid == 0)
        def _():
            pltpu.sync_copy(acc.at[pl.ds(0, _NPW), :], out0_hbm.at[pl.ds(r0, _NPW), :])

        @pl.when(cid == 1)
        def _():
            pltpu.sync_copy(acc.at[pl.ds(0, _NPW), :], out1_hbm.at[pl.ds(r0, _NPW), :])
    else:
        stg = accs_and_sems[nacc]

        def wblock(k, _):
            def srow(r, _):
                i = k * 64 + r
                rr = i >> shift
                cb = (i & lmask) * dc
                for j in range(nj):
                    v = accs[0][rr, pl.ds(cb + j * 16, 16)]
                    for a_t in accs[1:]:
                        v = jnp.minimum(v, a_t[rr, pl.ds(cb + j * 16, 16)])
                    stg[r, pl.ds(j * 16, 16)] = v
                return 0

            lax.fori_loop(0, 64, srow, 0)

            @pl.when(cid == 0)
            def _():
                pltpu.sync_copy(stg, out0_hbm.at[pl.ds(r0 + k * 64, 64), :])

            @pl.when(cid == 1)
            def _():
                pltpu.sync_copy(stg, out1_hbm.at[pl.ds(r0 + k * 64, 64), :])

            return 0

        lax.fori_loop(0, _NPW // 64, wblock, 0)


def _sc_segmin(b, gsrc, gld, meta, dc):
    nacc = 1 if dc > 64 else (2 if dc > 32 else 4)
    stg_scr = [] if nacc == 1 else [pltpu.VMEM((64, 128), F32)]
    return pl.kernel(
        functools.partial(_sc_segmin_body, dc, nacc),
        out_type=[
            jax.ShapeDtypeStruct((_NP, 128), F32),
            jax.ShapeDtypeStruct((_NP, 128), F32),
        ],
        mesh=_mesh,
        compiler_params=pltpu.CompilerParams(needs_layout_passes=False),
        scratch_types=[
            pltpu.VMEM((128,), I32),
            pltpu.VMEM((128,), I32),
            pltpu.VMEM((128,), I32),
            pltpu.VMEM((128,), I32),
            pltpu.VMEM((128, 128), F32),
            pltpu.VMEM((128, 128), F32),
            pltpu.VMEM((512,), I32),
        ] + [pltpu.VMEM((_NPW // (128 // dc) + 1, 128), F32)] * nacc + stg_scr + [
            pltpu.SemaphoreType.DMA,
            pltpu.SemaphoreType.DMA,
            pltpu.SemaphoreType.DMA,
            pltpu.SemaphoreType.DMA,
            pltpu.SemaphoreType.DMA,
            pltpu.SemaphoreType.DMA,
        ],
    )(b, gsrc, gld, meta)


def _tc_mlp_body(ef_ref, xs_ref, w0_ref, b0_ref, w1_ref, b1_ref, w2_ref,
                 b2_ref, w3_ref, b3_ref, w4_ref, b4_ref, m0_ref, m1_ref):
    h = jnp.dot(ef_ref[...], w0_ref[...], preferred_element_type=F32)
    h = jnp.maximum(h + b0_ref[0, :], 0.0)
    h = jnp.dot(h, w1_ref[...], preferred_element_type=F32)
    h = jnp.maximum(h + b1_ref[0, :], 0.0)
    h = jnp.dot(h, w2_ref[...], preferred_element_type=F32)
    h = jnp.maximum(h + b2_ref[0, :], 0.0)
    h = jnp.dot(h, w3_ref[...], preferred_element_type=F32)
    h = jnp.maximum(h + b3_ref[0, :], 0.0)
    xs = xs_ref[...]
    acc = jnp.zeros((h.shape[0], 256), F32)
    for d in range(7):
        z = jnp.dot(h, w4_ref[:, d * 256:(d + 1) * 256],
                    preferred_element_type=F32)
        z = jax.nn.sigmoid(z + b4_ref[0, d * 256:(d + 1) * 256])
        acc = acc + xs[:, d:d + 1] * z
    m0_ref[...] = acc[:, :128]
    m1_ref[...] = acc[:, 128:]


def _tc_mlp(ef8, xs, w0, b0, w1, b1, w2, b2, w3, b3, w4, b4):
    blk = 1000
    grid = _E // blk
    full = lambda s: pl.BlockSpec(s, lambda i: (0, 0))
    return pl.pallas_call(
        _tc_mlp_body,
        grid=(grid,),
        in_specs=[
            pl.BlockSpec((blk, 8), lambda i: (i, 0)),
            pl.BlockSpec((blk, 16), lambda i: (i, 0)),
            full((8, 32)), full((1, 32)),
            full((32, 64)), full((1, 64)),
            full((64, 128)), full((1, 128)),
            full((128, 256)), full((1, 256)),
            full((256, 1792)), full((1, 1792)),
        ],
        out_specs=[
            pl.BlockSpec((blk, 128), lambda i: (i, 0)),
            pl.BlockSpec((blk, 128), lambda i: (i, 0)),
        ],
        out_shape=[
            jax.ShapeDtypeStruct((_E, 128), F32),
            jax.ShapeDtypeStruct((_E, 128), F32),
        ],
    )(ef8, xs, w0, b0, w1, b1, w2, b2, w3, b3, w4, b4)


def _tc_head_body(s0_ref, s1_ref, deg0_ref, deg1_ref, nnb_ref, wa_ref, ca_ref,
                  wb_ref, a_ref, b_ref, degx_ref):
    deg = deg0_ref[:, 0:1] + deg1_ref[:, 0:1]
    h = jnp.concatenate([s0_ref[...], s1_ref[...]], axis=1) / jnp.maximum(deg, 1.0)
    h = h + nnb_ref[0, :]
    a_ref[...] = jnp.dot(h, wa_ref[...], preferred_element_type=F32) + ca_ref[0, :]
    b_ref[...] = jnp.dot(h, wb_ref[...], preferred_element_type=F32)
    degx_ref[...] = jnp.broadcast_to(deg, (deg.shape[0], 8))


def _tc_head(s0, s1, deg0, deg1, nnb, wa, ca, wb):
    blk = 1024
    grid = _NP // blk
    full = lambda s: pl.BlockSpec(s, lambda i: (0, 0))
    return pl.pallas_call(
        _tc_head_body,
        grid=(grid,),
        in_specs=[
            pl.BlockSpec((blk, 128), lambda i: (i, 0)),
            pl.BlockSpec((blk, 128), lambda i: (i, 0)),
            pl.BlockSpec((blk, 128), lambda i: (i, 0)),
            pl.BlockSpec((blk, 128), lambda i: (i, 0)),
            full((1, 256)),
            full((256, 128)), full((1, 128)),
            full((256, 128)),
        ],
        out_specs=[
            pl.BlockSpec((blk, 128), lambda i: (i, 0)),
            pl.BlockSpec((blk, 128), lambda i: (i, 0)),
            pl.BlockSpec((blk, 8), lambda i: (i, 0)),
        ],
        out_shape=[
            jax.ShapeDtypeStruct((_NP, 128), F32),
            jax.ShapeDtypeStruct((_NP, 128), F32),
            jax.ShapeDtypeStruct((_NP, 8), F32),
        ],
    )(s0, s1, deg0, deg1, nnb, wa, ca, wb)


def _tc_step_body(din, dout, a_ref, m0_ref, m1_ref, degx_ref, wa_ref, ca_ref,
                  wb_ref, an_ref, b_ref):
    blk = a_ref.shape[0]
    degc = degx_ref[:, 0:1]
    m = jnp.minimum(m0_ref[...], m1_ref[...])[:, :din]
    h = jnp.where(degc > 0.0, a_ref[:, :din] - m, 0.0)
    a = jnp.dot(h, wa_ref[...], preferred_element_type=F32) + ca_ref[0, :]
    b = jnp.dot(h, wb_ref[...], preferred_element_type=F32)
    if dout < 128:
        pad = jnp.zeros((blk, 128 - dout), F32)
        a = jnp.concatenate([a, pad], axis=1)
        b = jnp.concatenate([b, pad], axis=1)
    an_ref[...] = a
    b_ref[...] = b


def _tc_step(a, m0, m1, degx, wa, ca, wb):
    blk = 1024
    grid = _NP // blk
    din = wa.shape[0]
    dout = wa.shape[1]
    full = lambda s: pl.BlockSpec(s, lambda i: (0, 0))
    return pl.pallas_call(
        functools.partial(_tc_step_body, din, dout),
        grid=(grid,),
        in_specs=[
            pl.BlockSpec((blk, 128), lambda i: (i, 0)),
            pl.BlockSpec((blk, 128), lambda i: (i, 0)),
            pl.BlockSpec((blk, 128), lambda i: (i, 0)),
            pl.BlockSpec((blk, 8), lambda i: (i, 0)),
            full((din, dout)), full((1, dout)),
            full((din, dout)),
        ],
        out_specs=[
            pl.BlockSpec((blk, 128), lambda i: (i, 0)),
            pl.BlockSpec((blk, 128), lambda i: (i, 0)),
        ],
        out_shape=[
            jax.ShapeDtypeStruct((_NP, 128), F32),
            jax.ShapeDtypeStruct((_NP, 128), F32),
        ],
    )(a, m0, m1, degx, wa, ca, wb)


def _tc_final_body(din, a_ref, m0_ref, m1_ref, degx_ref, h_ref):
    degc = degx_ref[:, 0:1]
    m = jnp.minimum(m0_ref[...], m1_ref[...])[:, :din]
    h_ref[...] = jnp.where(degc > 0.0, a_ref[:, :din] - m, 0.0)


def _tc_final(a, m0, m1, degx, din):
    blk = 1024
    grid = _NP // blk
    return pl.pallas_call(
        functools.partial(_tc_final_body, din),
        grid=(grid,),
        in_specs=[
            pl.BlockSpec((blk, 128), lambda i: (i, 0)),
            pl.BlockSpec((blk, 128), lambda i: (i, 0)),
            pl.BlockSpec((blk, 128), lambda i: (i, 0)),
            pl.BlockSpec((blk, 8), lambda i: (i, 0)),
        ],
        out_specs=pl.BlockSpec((blk, din), lambda i: (i, 0)),
        out_shape=jax.ShapeDtypeStruct((_NP, din), F32),
    )(a, m0, m1, degx)


def kernel(x, edge_feat, edge_index, params):
    src = edge_index[0]
    dst = edge_index[1]

    s_bn = np.float32(1.0) / np.sqrt(np.float32(1.0) + np.float32(_EPS))

    mlp = params["mlp"]
    ws, bs = [], []
    for i in range(4):
        g = mlp[f"g{i}"] * s_bn
        ws.append(mlp[f"W{i}"] * g[None, :])
        bs.append((mlp[f"b{i}"] * g + mlp[f"be{i}"])[None, :])
    w0 = jnp.zeros((8, 32), F32).at[:3].set(ws[0])
    w4 = mlp["W4"]
    b4 = mlp["b4"][None, :]

    ecw = []
    for p in params["ecs"]:
        sg = p["g"] * s_bn
        wa = (p["tW"] + p["pW"]) * sg[None, :]
        wb = p["tW"] * sg[None, :]
        c = ((p["tb"] + p["pb"]) * sg + p["be"])[None, :]
        ecw.append((wa, c, wb))

    xp128 = jnp.zeros((_NP, 128), F32).at[:_N, :7].set(x)
    ef8 = jnp.zeros((_E, 8), F32).at[:, :3].set(edge_feat)

    xs = _sc_gather(xp128, src)

    msg0, msg1 = _tc_mlp(ef8, xs, w0, bs[0], ws[1], bs[1], ws[2], bs[2],
                         ws[3], bs[3], w4, b4)

    sum0, sum1, deg0, deg1 = _sc_segsum(msg0, msg1, dst)

    gsrc, gld, meta = _sc_partition(src, dst)

    nnb = params["nn_bias"][None, :]
    wa, ca, wb = ecw[0]
    a, b, degx = _tc_head(sum0, sum1, deg0, deg1, nnb, wa, ca, wb)

    dl = 128
    for l in (1, 2, 3):
        m0, m1 = _sc_segmin(b, gsrc, gld, meta, dl)
        wa, ca, wb = ecw[l]
        a, b = _tc_step(a, m0, m1, degx, wa, ca, wb)
        dl = wa.shape[1]

    m0, m1 = _sc_segmin(b, gsrc, gld, meta, dl)
    h = _tc_final(a, m0, m1, degx, dl)
    return h[:_N]


# speedup vs baseline: 1.0155x; 1.0155x over previous
import functools

import numpy as np

import jax
import jax.numpy as jnp
from jax import lax
from jax.experimental import pallas as pl
from jax.experimental.pallas import tpu as pltpu
from jax.experimental.pallas import tpu_sc as plsc

F32 = jnp.float32
I32 = jnp.int32

_N = 10000
_E = 160000
_NP = 10240
_NPW = 640
_DUMMY = _NPW
_EPS = 1e-5
_INF = np.float32(3.0e38)

_EH = _E // 2
_CH = 8000
_NCHUNK = _EH // _CH
_FLUSH = _CH + 16
_CAPR = 88576

_mesh = plsc.VectorSubcoreMesh(core_axis_name="c", subcore_axis_name="s")


def _sc_gather_body(xp_hbm, src_hbm, xs_hbm, srcb0, srcb1, srcb16, rows0,
                    rows1, xsb0, xsb1, ss0, ss1, sg0, sg1, so0, so1):
    wid = lax.axis_index("s") * 2 + lax.axis_index("c")
    base = wid * (_E // 32)
    nb = 39

    srcbs = (srcb0, srcb1)
    rowss = (rows0, rows1)
    xsbs = (xsb0, xsb1)
    sss = (ss0, ss1)
    sgs = (sg0, sg1)
    sos = (so0, so1)

    def s_src(b, slot):
        pltpu.async_copy(src_hbm.at[pl.ds(base + b * 128, 128)], srcbs[slot], sss[slot])

    def w_src(slot):
        pltpu.make_async_copy(src_hbm.at[pl.ds(base, 128)], srcbs[slot], sss[slot]).wait()

    def s_g(slot):
        pltpu.async_copy(xp_hbm.at[srcbs[slot]], rowss[slot], sgs[slot])

    def w_g(slot):
        pltpu.make_async_copy(xp_hbm.at[srcbs[slot]], rowss[slot], sgs[slot]).wait()

    def s_out(b, slot):
        pltpu.async_copy(xsbs[slot], xs_hbm.at[pl.ds(base + b * 128, 128), :], sos[slot])

    def w_out(slot):
        pltpu.make_async_copy(xsbs[slot], xs_hbm.at[pl.ds(base, 128), :], sos[slot]).wait()

    s_src(0, 0)
    w_src(0)
    s_g(0)
    s_src(1, 1)

    def pairloop(g, _):
        for slot in (0, 1):
            b = g * 2 + slot

            @pl.when(b < nb)
            def _():
                @pl.when(b + 1 < nb)
                def _():
                    w_src((slot + 1) % 2)
                    s_g((slot + 1) % 2)

                w_g(slot)

                @pl.when(b + 2 < nb)
                def _():
                    s_src(b + 2, slot)

                @pl.when(b >= 2)
                def _():
                    w_out(slot)

                rows = rowss[slot]
                xsb = xsbs[slot]

                def crow(r, _):
                    xsb[r, :] = rows[r, pl.ds(0, 16)]
                    return 0

                lax.fori_loop(0, 128, crow, 0)
                s_out(b, slot)
        return 0

    lax.fori_loop(0, (nb + 1) // 2, pairloop, 0)
    w_out((nb - 2) % 2)
    w_out((nb - 1) % 2)

    srcb16[:] = jnp.zeros((16,), I32)
    e0 = base + 39 * 128
    pltpu.sync_copy(src_hbm.at[pl.ds(e0, 8)], srcb16.at[pl.ds(0, 8)])
    pltpu.async_copy(xp_hbm.at[srcb16], rows0.at[pl.ds(0, 16), :], sg0).wait()

    def crow8(r, _):
        xsb0[r, :] = rows0[r, pl.ds(0, 16)]
        return 0

    lax.fori_loop(0, 8, crow8, 0)
    pltpu.sync_copy(xsb0.at[pl.ds(0, 8), :], xs_hbm.at[pl.ds(e0, 8), :])


def _sc_gather(xp128, src):
    return pl.kernel(
        _sc_gather_body,
        out_type=jax.ShapeDtypeStruct((_E, 16), F32),
        mesh=_mesh,
        compiler_params=pltpu.CompilerParams(needs_layout_passes=False),
        scratch_types=[
            pltpu.VMEM((128,), I32),
            pltpu.VMEM((128,), I32),
            pltpu.VMEM((16,), I32),
            pltpu.VMEM((128, 128), F32),
            pltpu.VMEM((128, 128), F32),
            pltpu.VMEM((128, 16), F32),
            pltpu.VMEM((128, 16), F32),
            pltpu.SemaphoreType.DMA,
            pltpu.SemaphoreType.DMA,
            pltpu.SemaphoreType.DMA,
            pltpu.SemaphoreType.DMA,
            pltpu.SemaphoreType.DMA,
            pltpu.SemaphoreType.DMA,
        ],
    )(xp128, src)


def _sc_segsum_body(msg0_hbm, msg1_hbm, dst_hbm, sum0_hbm, sum1_hbm,
                    deg0_hbm, deg1_hbm, idxb0, idxb1, idxb2, idxb40,
                    rows0, rows1, rows2, ones, zbuf, S,
                    sd0, sd1, sd2, sm0, sm1, sm2, sc0, sc1, sc2):
    cid = lax.axis_index("c")
    sid = lax.axis_index("s")

    idxbs = (idxb0, idxb1, idxb2)
    rowss = (rows0, rows1, rows2)
    sds = (sd0, sd1, sd2)
    sms = (sm0, sm1, sm2)
    scs = (sc0, sc1, sc2)

    z16 = jnp.zeros((16,), F32)
    one16 = jnp.ones((16,), F32)

    def zrow(i, _):
        for j in range(8):
            zbuf[i, pl.ds(j * 16, 16)] = z16
        return 0

    lax.fori_loop(0, 32, zrow, 0)

    def orow(i, _):
        for j in range(8):
            ones[i, pl.ds(j * 16, 16)] = one16
        return 0

    lax.fori_loop(0, 80, orow, 0)

    def zs(k, _):
        pltpu.sync_copy(zbuf, S.at[pl.ds(sid * _NPW + k * 32, 32), :])
        return 0

    lax.fori_loop(0, _NPW // 32, zs, 0)

    plsc.subcore_barrier()

    base = sid * (_E // 16)
    nb = 125

    def s_ld(b, slot):
        e0 = base + b * 80
        pltpu.async_copy(dst_hbm.at[pl.ds(e0, 80)], idxbs[slot], sds[slot])

        @pl.when(cid == 0)
        def _():
            pltpu.async_copy(msg0_hbm.at[pl.ds(e0, 80), :], rowss[slot], sms[slot])

        @pl.when(cid == 1)
        def _():
            pltpu.async_copy(msg1_hbm.at[pl.ds(e0, 80), :], rowss[slot], sms[slot])

    def w_ld(slot):
        pltpu.make_async_copy(dst_hbm.at[pl.ds(base, 80)], idxbs[slot], sds[slot]).wait()
        pltpu.make_async_copy(msg0_hbm.at[pl.ds(base, 80), :], rowss[slot], sms[slot]).wait()

    def s_sc(slot):
        pltpu.async_copy(rowss[slot], S.at[idxbs[slot]], scs[slot], add=True)

    def w_sc(slot):
        pltpu.make_async_copy(rowss[slot], S.at[idxbs[slot]], scs[slot]).wait()

    s_ld(0, 0)
    s_ld(1, 1)

    def triloop(g, _):
        for slot in (0, 1, 2):
            b = g * 3 + slot

            @pl.when(b < nb)
            def _():
                w_ld(slot)
                s_sc(slot)
                prev = (slot + 2) % 3

                @pl.when(b >= 1)
                def _():
                    w_sc(prev)

                @pl.when(b + 2 < nb)
                def _():
                    s_ld(b + 2, prev)
        return 0

    lax.fori_loop(0, (nb + 2) // 3, triloop, 0)
    w_sc((nb - 1) % 3)

    plsc.subcore_barrier()

    r0 = sid * _NPW

    @pl.when(cid == 0)
    def _():
        pltpu.sync_copy(S.at[pl.ds(r0, _NPW), :], sum0_hbm.at[pl.ds(r0, _NPW), :])

    @pl.when(cid == 1)
    def _():
        pltpu.sync_copy(S.at[pl.ds(r0, _NPW), :], sum1_hbm.at[pl.ds(r0, _NPW), :])

    plsc.subcore_barrier()

    def zs2(k, _):
        pltpu.sync_copy(zbuf, S.at[pl.ds(sid * _NPW + k * 32, 32), :])
        return 0

    lax.fori_loop(0, _NPW // 32, zs2, 0)
    plsc.subcore_barrier()

    dbase = cid * _EH + sid * (_EH // 16)
    nd = 62

    def d_ld(b, slot):
        pltpu.async_copy(dst_hbm.at[pl.ds(dbase + b * 80, 80)], idxbs[slot], sds[slot])

    def dw_ld(slot):
        pltpu.make_async_copy(dst_hbm.at[pl.ds(dbase, 80)], idxbs[slot], sds[slot]).wait()

    def d_sc(slot):
        pltpu.async_copy(ones, S.at[idxbs[slot]], scs[slot], add=True)

    def dw_sc(slot):
        pltpu.make_async_copy(ones, S.at[idxbs[slot]], scs[slot]).wait()

    d_ld(0, 0)
    d_ld(1, 1)

    def dtriloop(g, _):
        for slot in (0, 1, 2):
            b = g * 3 + slot

            @pl.when(b < nd)
            def _():
                dw_ld(slot)
                d_sc(slot)
                prev = (slot + 2) % 3

                @pl.when(b >= 1)
                def _():
                    dw_sc(prev)

                @pl.when(b + 2 < nd)
                def _():
                    d_ld(b + 2, prev)
        return 0

    lax.fori_loop(0, (nd + 2) // 3, dtriloop, 0)
    dw_sc((nd - 1) % 3)

    pltpu.sync_copy(dst_hbm.at[pl.ds(dbase + 4960, 40)], idxb40)
    pltpu.sync_copy(ones.at[pl.ds(0, 40), :], S.at[idxb40], add=True)

    plsc.subcore_barrier()

    @pl.when(cid == 0)
    def _():
        pltpu.sync_copy(S.at[pl.ds(r0, _NPW), :], deg0_hbm.at[pl.ds(r0, _NPW), :])

    @pl.when(cid == 1)
    def _():
        pltpu.sync_copy(S.at[pl.ds(r0, _NPW), :], deg1_hbm.at[pl.ds(r0, _NPW), :])


def _sc_segsum(msg0, msg1, dst):
    return pl.kernel(
        _sc_segsum_body,
        out_type=[
            jax.ShapeDtypeStruct((_NP, 128), F32),
            jax.ShapeDtypeStruct((_NP, 128), F32),
            jax.ShapeDtypeStruct((_NP, 128), F32),
            jax.ShapeDtypeStruct((_NP, 128), F32),
        ],
        mesh=_mesh,
        compiler_params=pltpu.CompilerParams(needs_layout_passes=False),
        scratch_types=[
            pltpu.VMEM((80,), I32),
            pltpu.VMEM((80,), I32),
            pltpu.VMEM((80,), I32),
            pltpu.VMEM((40,), I32),
            pltpu.VMEM((80, 128), F32),
            pltpu.VMEM((80, 128), F32),
            pltpu.VMEM((80, 128), F32),
            pltpu.VMEM((80, 128), F32),
            pltpu.VMEM((32, 128), F32),
            pltpu.VMEM_SHARED((_NP, 128), F32),
            pltpu.SemaphoreType.DMA,
            pltpu.SemaphoreType.DMA,
            pltpu.SemaphoreType.DMA,
            pltpu.SemaphoreType.DMA,
            pltpu.SemaphoreType.DMA,
            pltpu.SemaphoreType.DMA,
            pltpu.SemaphoreType.DMA,
            pltpu.SemaphoreType.DMA,
            pltpu.SemaphoreType.DMA,
        ],
    )(msg0, msg1, dst)


def _sc_part_body(src_hbm, dst_hbm, gsrc_hbm, gld_hbm, meta_hbm,
                  srcv0, srcv1, dstv0, dstv1, bufS0, bufS1, bufL0, bufL1,
                  dumS, dumL, metab, ss0, ss1, sd0, sd1, fS0, fS1, fL0, fL1):
    cid = lax.axis_index("c")
    sid = lax.axis_index("s")
    row = cid * 16 + sid
    ebase = cid * _EH
    lo = sid * _NPW

    srcvs = (srcv0, srcv1)
    dstvs = (dstv0, dstv1)
    bufSs = (bufS0, bufS1)
    bufLs = (bufL0, bufL1)
    sss = (ss0, ss1)
    sds = (sd0, sd1)
    fSs = (fS0, fS1)
    fLs = (fL0, fL1)

    z16 = jnp.zeros((16,), I32)
    d16 = jnp.full((16,), _DUMMY, I32)

    def fillbuf(i, _):
        bufS0[pl.ds(i * 16, 16)] = z16
        bufS1[pl.ds(i * 16, 16)] = z16
        bufL0[pl.ds(i * 16, 16)] = d16
        bufL1[pl.ds(i * 16, 16)] = d16
        return 0

    lax.fori_loop(0, _FLUSH // 16, fillbuf, 0)

    def filldum(i, _):
        dumS[pl.ds(i * 16, 16)] = z16
        dumL[pl.ds(i * 16, 16)] = d16
        return 0

    lax.fori_loop(0, 8, filldum, 0)

    def s_in(c, slot):
        pltpu.async_copy(src_hbm.at[pl.ds(ebase + c * _CH, _CH)], srcvs[slot], sss[slot])
        pltpu.async_copy(dst_hbm.at[pl.ds(ebase + c * _CH, _CH)], dstvs[slot], sds[slot])

    def w_in(slot):
        pltpu.make_async_copy(src_hbm.at[pl.ds(ebase, _CH)], srcvs[slot], sss[slot]).wait()
        pltpu.make_async_copy(dst_hbm.at[pl.ds(ebase, _CH)], dstvs[slot], sds[slot]).wait()

    def s_fl(slot, written):
        wo = pl.multiple_of(row * _CAPR + written, 16)
        pltpu.async_copy(bufSs[slot], gsrc_hbm.at[pl.ds(wo, _FLUSH)], fSs[slot])
        pltpu.async_copy(bufLs[slot], gld_hbm.at[pl.ds(wo, _FLUSH)], fLs[slot])

    def w_fl(slot):
        pltpu.make_async_copy(bufSs[slot], gsrc_hbm.at[pl.ds(0, _FLUSH)], fSs[slot]).wait()
        pltpu.make_async_copy(bufLs[slot], gld_hbm.at[pl.ds(0, _FLUSH)], fLs[slot]).wait()

    s_in(0, 0)
    s_in(1, 1)

    def pair(g, written):
        for slot in (0, 1):
            c = g * 2 + slot
            w_in(slot)

            @pl.when(c >= 2)
            def _():
                w_fl(slot)

            srcv = srcvs[slot]
            dstv = dstvs[slot]
            bufS = bufSs[slot]
            bufL = bufLs[slot]

            def scan(i, off):
                d = dstv[pl.ds(i * 16, 16)]
                sv = srcv[pl.ds(i * 16, 16)]
                m = (d >= lo) & (d < lo + _NPW)
                plsc.store_compressed(bufS.at[pl.ds(off, 16)], sv, mask=m)
                plsc.store_compressed(bufL.at[pl.ds(off, 16)], d - lo, mask=m)
                return off + jnp.sum(m.astype(I32))

            off = lax.fori_loop(0, _CH // 16, scan, 0)
            bufS[pl.ds(off, 16)] = z16
            bufL[pl.ds(off, 16)] = d16
            off16 = (off + 15) & ~15
            s_fl(slot, written)

            @pl.when(c + 2 < _NCHUNK)
            def _():
                s_in(c + 2, slot)

            written = written + off16
        return written

    written = lax.fori_loop(0, _NCHUNK // 2, pair, 0)
    w_fl(0)
    w_fl(1)
    wo = pl.multiple_of(row * _CAPR + written, 16)
    pltpu.sync_copy(dumS, gsrc_hbm.at[pl.ds(wo, 128)])
    pltpu.sync_copy(dumL, gld_hbm.at[pl.ds(wo, 128)])
    m128 = (written + 127) & ~127
    metab[:] = jnp.zeros((16,), I32) + m128
    pltpu.sync_copy(metab, meta_hbm.at[pl.ds(pl.multiple_of(row * 16, 16), 16)])


def _sc_partition(src, dst):
    return pl.kernel(
        _sc_part_body,
        out_type=[
            jax.ShapeDtypeStruct((32 * _CAPR,), I32),
            jax.ShapeDtypeStruct((32 * _CAPR,), I32),
            jax.ShapeDtypeStruct((512,), I32),
        ],
        mesh=_mesh,
        compiler_params=pltpu.CompilerParams(needs_layout_passes=False),
        scratch_types=[
            pltpu.VMEM((_CH,), I32),
            pltpu.VMEM((_CH,), I32),
            pltpu.VMEM((_CH,), I32),
            pltpu.VMEM((_CH,), I32),
            pltpu.VMEM((_FLUSH,), I32),
            pltpu.VMEM((_FLUSH,), I32),
            pltpu.VMEM((_FLUSH,), I32),
            pltpu.VMEM((_FLUSH,), I32),
            pltpu.VMEM((128,), I32),
            pltpu.VMEM((128,), I32),
            pltpu.VMEM((16,), I32),
            pltpu.SemaphoreType.DMA,
            pltpu.SemaphoreType.DMA,
            pltpu.SemaphoreType.DMA,
            pltpu.SemaphoreType.DMA,
            pltpu.SemaphoreType.DMA,
            pltpu.SemaphoreType.DMA,
            pltpu.SemaphoreType.DMA,
            pltpu.SemaphoreType.DMA,
        ],
    )(src, dst)


def _sc_segmin_body(dc, nacc, b_hbm, gsrc_hbm, gld_hbm, meta_hbm, out0_hbm,
                    out1_hbm, idxb0, idxb1, ldv0, ldv1, rows0, rows1, metav,
                    *accs_and_sems):
    accs = accs_and_sems[:nacc]
    si0, si1, sl0, sl1, sg0, sg1 = accs_and_sems[-6:]
    acc = accs[0]
    cid = lax.axis_index("c")
    sid = lax.axis_index("s")
    row = cid * 16 + sid
    nj = dc // 16
    pf = 1 if nacc == 1 else 128 // dc
    shift = {1: 0, 2: 1, 4: 2}[pf]
    lmask = pf - 1

    idxbs = (idxb0, idxb1)
    ldvs = (ldv0, ldv1)
    rowss = (rows0, rows1)
    sis = (si0, si1)
    sls = (sl0, sl1)
    sgs = (sg0, sg1)

    inf16 = jnp.full((16,), _INF, F32)

    def initrow(i, _):
        for a in accs:
            for j in range(8):
                a[i, pl.ds(j * 16, 16)] = inf16
        return 0

    lax.fori_loop(0, _NPW // pf + 1, initrow, 0)

    pltpu.sync_copy(meta_hbm, metav)
    m128 = jnp.max(metav[pl.ds(pl.multiple_of(row * 16, 16), 16)])
    nb = m128 // 128

    def s_idx(b, slot):
        bo = pl.multiple_of(row * _CAPR + b * 128, 128)
        pltpu.async_copy(gsrc_hbm.at[pl.ds(bo, 128)], idxbs[slot], sis[slot])
        pltpu.async_copy(gld_hbm.at[pl.ds(bo, 128)], ldvs[slot], sls[slot])

    def w_idx(slot):
        pltpu.make_async_copy(gsrc_hbm.at[pl.ds(0, 128)], idxbs[slot], sis[slot]).wait()
        pltpu.make_async_copy(gld_hbm.at[pl.ds(0, 128)], ldvs[slot], sls[slot]).wait()

    def s_g(slot):
        pltpu.async_copy(b_hbm.at[idxbs[slot]], rowss[slot], sgs[slot])

    def w_g(slot):
        pltpu.make_async_copy(b_hbm.at[idxbs[slot]], rowss[slot], sgs[slot]).wait()

    def upd(slot):
        ldv = ldvs[slot]
        rows = rowss[slot]

        def qloop(q, _):
            ld16 = ldv[pl.ds(q * 16, 16)]
            for i in range(16):
                ld = ld16[i]
                t = q * 16 + i
                a_t = accs[i % nacc]
                if pf == 1:
                    rr = ld
                    cb = 0
                else:
                    rr = ld >> shift
                    cb = (ld & lmask) * dc
                for j in range(nj):
                    a = a_t[rr, pl.ds(cb + j * 16, 16)]
                    r = rows[t, pl.ds(j * 16, 16)]
                    a_t[rr, pl.ds(cb + j * 16, 16)] = jnp.minimum(a, r)
            return 0

        lax.fori_loop(0, 8, qloop, 0)

    @pl.when(nb > 0)
    def _():
        s_idx(0, 0)
        w_idx(0)
        s_g(0)

        @pl.when(nb > 1)
        def _():
            s_idx(1, 1)

    def pairloop(g, _):
        for slot in (0, 1):
            b = g * 2 + slot

            @pl.when(b < nb)
            def _():
                @pl.when(b + 1 < nb)
                def _():
                    w_idx((slot + 1) % 2)
                    s_g((slot + 1) % 2)

                w_g(slot)
                upd(slot)

                @pl.when(b + 2 < nb)
                def _():
                    s_idx(b + 2, slot)
        return 0

    lax.fori_loop(0, (nb + 1) // 2, pairloop, 0)

    r0 = sid * _NPW

    if nacc == 1:
        @pl.when(cid == 0)
        def _():
            pltpu.sync_copy(acc.at[pl.ds(0, _NPW), :], out0_hbm.at[pl.ds(r0, _NPW), :])

        @pl.when(cid == 1)
        def _():
            pltpu.sync_copy(acc.at[pl.ds(0, _NPW), :], out1_hbm.at[pl.ds(r0, _NPW), :])
    else:
        stg = accs_and_sems[nacc]

        def wblock(k, _):
            def srow(r, _):
                i = k * 64 + r
                rr = i >> shift
                cb = (i & lmask) * dc
                for j in range(nj):
                    v = accs[0][rr, pl.ds(cb + j * 16, 16)]
                    for a_t in accs[1:]:
                        v = jnp.minimum(v, a_t[rr, pl.ds(cb + j * 16, 16)])
                    stg[r, pl.ds(j * 16, 16)] = v
                return 0

            lax.fori_loop(0, 64, srow, 0)

            @pl.when(cid == 0)
            def _():
                pltpu.sync_copy(stg, out0_hbm.at[pl.ds(r0 + k * 64, 64), :])

            @pl.when(cid == 1)
            def _():
                pltpu.sync_copy(stg, out1_hbm.at[pl.ds(r0 + k * 64, 64), :])

            return 0

        lax.fori_loop(0, _NPW // 64, wblock, 0)


def _sc_segmin(b, gsrc, gld, meta, dc):
    nacc = 1
    stg_scr = [] if nacc == 1 else [pltpu.VMEM((64, 128), F32)]
    return pl.kernel(
        functools.partial(_sc_segmin_body, dc, nacc),
        out_type=[
            jax.ShapeDtypeStruct((_NP, 128), F32),
            jax.ShapeDtypeStruct((_NP, 128), F32),
        ],
        mesh=_mesh,
        compiler_params=pltpu.CompilerParams(needs_layout_passes=False),
        scratch_types=[
            pltpu.VMEM((128,), I32),
            pltpu.VMEM((128,), I32),
            pltpu.VMEM((128,), I32),
            pltpu.VMEM((128,), I32),
            pltpu.VMEM((128, 128), F32),
            pltpu.VMEM((128, 128), F32),
            pltpu.VMEM((512,), I32),
        ] + [pltpu.VMEM((_NPW // (128 // dc if nacc > 1 else 1) + 1, 128), F32)] * nacc + stg_scr + [
            pltpu.SemaphoreType.DMA,
            pltpu.SemaphoreType.DMA,
            pltpu.SemaphoreType.DMA,
            pltpu.SemaphoreType.DMA,
            pltpu.SemaphoreType.DMA,
            pltpu.SemaphoreType.DMA,
        ],
    )(b, gsrc, gld, meta)


def _tc_mlp_body(ef_ref, xs_ref, w0_ref, b0_ref, w1_ref, b1_ref, w2_ref,
                 b2_ref, w3_ref, b3_ref, w4_ref, b4_ref, m0_ref, m1_ref):
    h = jnp.dot(ef_ref[...], w0_ref[...], preferred_element_type=F32)
    h = jnp.maximum(h + b0_ref[0, :], 0.0)
    h = jnp.dot(h, w1_ref[...], preferred_element_type=F32)
    h = jnp.maximum(h + b1_ref[0, :], 0.0)
    h = jnp.dot(h, w2_ref[...], preferred_element_type=F32)
    h = jnp.maximum(h + b2_ref[0, :], 0.0)
    h = jnp.dot(h, w3_ref[...], preferred_element_type=F32)
    h = jnp.maximum(h + b3_ref[0, :], 0.0)
    xs = xs_ref[...]
    acc = jnp.zeros((h.shape[0], 256), F32)
    for d in range(7):
        z = jnp.dot(h, w4_ref[:, d * 256:(d + 1) * 256],
                    preferred_element_type=F32)
        z = jax.nn.sigmoid(z + b4_ref[0, d * 256:(d + 1) * 256])
        acc = acc + xs[:, d:d + 1] * z
    m0_ref[...] = acc[:, :128]
    m1_ref[...] = acc[:, 128:]


def _tc_mlp(ef8, xs, w0, b0, w1, b1, w2, b2, w3, b3, w4, b4):
    blk = 1000
    grid = _E // blk
    full = lambda s: pl.BlockSpec(s, lambda i: (0, 0))
    return pl.pallas_call(
        _tc_mlp_body,
        grid=(grid,),
        in_specs=[
            pl.BlockSpec((blk, 8), lambda i: (i, 0)),
            pl.BlockSpec((blk, 16), lambda i: (i, 0)),
            full((8, 32)), full((1, 32)),
            full((32, 64)), full((1, 64)),
            full((64, 128)), full((1, 128)),
            full((128, 256)), full((1, 256)),
            full((256, 1792)), full((1, 1792)),
        ],
        out_specs=[
            pl.BlockSpec((blk, 128), lambda i: (i, 0)),
            pl.BlockSpec((blk, 128), lambda i: (i, 0)),
        ],
        out_shape=[
            jax.ShapeDtypeStruct((_E, 128), F32),
            jax.ShapeDtypeStruct((_E, 128), F32),
        ],
    )(ef8, xs, w0, b0, w1, b1, w2, b2, w3, b3, w4, b4)


def _tc_head_body(s0_ref, s1_ref, deg0_ref, deg1_ref, nnb_ref, wa_ref, ca_ref,
                  wb_ref, a_ref, b_ref, degx_ref):
    deg = deg0_ref[:, 0:1] + deg1_ref[:, 0:1]
    h = jnp.concatenate([s0_ref[...], s1_ref[...]], axis=1) / jnp.maximum(deg, 1.0)
    h = h + nnb_ref[0, :]
    a_ref[...] = jnp.dot(h, wa_ref[...], preferred_element_type=F32) + ca_ref[0, :]
    b_ref[...] = jnp.dot(h, wb_ref[...], preferred_element_type=F32)
    degx_ref[...] = jnp.broadcast_to(deg, (deg.shape[0], 8))


def _tc_head(s0, s1, deg0, deg1, nnb, wa, ca, wb):
    blk = 1024
    grid = _NP // blk
    full = lambda s: pl.BlockSpec(s, lambda i: (0, 0))
    return pl.pallas_call(
        _tc_head_body,
        grid=(grid,),
        in_specs=[
            pl.BlockSpec((blk, 128), lambda i: (i, 0)),
            pl.BlockSpec((blk, 128), lambda i: (i, 0)),
            pl.BlockSpec((blk, 128), lambda i: (i, 0)),
            pl.BlockSpec((blk, 128), lambda i: (i, 0)),
            full((1, 256)),
            full((256, 128)), full((1, 128)),
            full((256, 128)),
        ],
        out_specs=[
            pl.BlockSpec((blk, 128), lambda i: (i, 0)),
            pl.BlockSpec((blk, 128), lambda i: (i, 0)),
            pl.BlockSpec((blk, 8), lambda i: (i, 0)),
        ],
        out_shape=[
            jax.ShapeDtypeStruct((_NP, 128), F32),
            jax.ShapeDtypeStruct((_NP, 128), F32),
            jax.ShapeDtypeStruct((_NP, 8), F32),
        ],
    )(s0, s1, deg0, deg1, nnb, wa, ca, wb)


def _tc_step_body(din, dout, a_ref, m0_ref, m1_ref, degx_ref, wa_ref, ca_ref,
                  wb_ref, an_ref, b_ref):
    blk = a_ref.shape[0]
    degc = degx_ref[:, 0:1]
    m = jnp.minimum(m0_ref[...], m1_ref[...])[:, :din]
    h = jnp.where(degc > 0.0, a_ref[:, :din] - m, 0.0)
    a = jnp.dot(h, wa_ref[...], preferred_element_type=F32) + ca_ref[0, :]
    b = jnp.dot(h, wb_ref[...], preferred_element_type=F32)
    if dout < 128:
        pad = jnp.zeros((blk, 128 - dout), F32)
        a = jnp.concatenate([a, pad], axis=1)
        b = jnp.concatenate([b, pad], axis=1)
    an_ref[...] = a
    b_ref[...] = b


def _tc_step(a, m0, m1, degx, wa, ca, wb):
    blk = 1024
    grid = _NP // blk
    din = wa.shape[0]
    dout = wa.shape[1]
    full = lambda s: pl.BlockSpec(s, lambda i: (0, 0))
    return pl.pallas_call(
        functools.partial(_tc_step_body, din, dout),
        grid=(grid,),
        in_specs=[
            pl.BlockSpec((blk, 128), lambda i: (i, 0)),
            pl.BlockSpec((blk, 128), lambda i: (i, 0)),
            pl.BlockSpec((blk, 128), lambda i: (i, 0)),
            pl.BlockSpec((blk, 8), lambda i: (i, 0)),
            full((din, dout)), full((1, dout)),
            full((din, dout)),
        ],
        out_specs=[
            pl.BlockSpec((blk, 128), lambda i: (i, 0)),
            pl.BlockSpec((blk, 128), lambda i: (i, 0)),
        ],
        out_shape=[
            jax.ShapeDtypeStruct((_NP, 128), F32),
            jax.ShapeDtypeStruct((_NP, 128), F32),
        ],
    )(a, m0, m1, degx, wa, ca, wb)


def _tc_final_body(din, a_ref, m0_ref, m1_ref, degx_ref, h_ref):
    degc = degx_ref[:, 0:1]
    m = jnp.minimum(m0_ref[...], m1_ref[...])[:, :din]
    h_ref[...] = jnp.where(degc > 0.0, a_ref[:, :din] - m, 0.0)


def _tc_final(a, m0, m1, degx, din):
    blk = 1024
    grid = _NP // blk
    return pl.pallas_call(
        functools.partial(_tc_final_body, din),
        grid=(grid,),
        in_specs=[
            pl.BlockSpec((blk, 128), lambda i: (i, 0)),
            pl.BlockSpec((blk, 128), lambda i: (i, 0)),
            pl.BlockSpec((blk, 128), lambda i: (i, 0)),
            pl.BlockSpec((blk, 8), lambda i: (i, 0)),
        ],
        out_specs=pl.BlockSpec((blk, din), lambda i: (i, 0)),
        out_shape=jax.ShapeDtypeStruct((_NP, din), F32),
    )(a, m0, m1, degx)


def kernel(x, edge_feat, edge_index, params):
    src = edge_index[0]
    dst = edge_index[1]

    s_bn = np.float32(1.0) / np.sqrt(np.float32(1.0) + np.float32(_EPS))

    mlp = params["mlp"]
    ws, bs = [], []
    for i in range(4):
        g = mlp[f"g{i}"] * s_bn
        ws.append(mlp[f"W{i}"] * g[None, :])
        bs.append((mlp[f"b{i}"] * g + mlp[f"be{i}"])[None, :])
    w0 = jnp.zeros((8, 32), F32).at[:3].set(ws[0])
    w4 = mlp["W4"]
    b4 = mlp["b4"][None, :]

    ecw = []
    for p in params["ecs"]:
        sg = p["g"] * s_bn
        wa = (p["tW"] + p["pW"]) * sg[None, :]
        wb = p["tW"] * sg[None, :]
        c = ((p["tb"] + p["pb"]) * sg + p["be"])[None, :]
        ecw.append((wa, c, wb))

    xp128 = jnp.zeros((_NP, 128), F32).at[:_N, :7].set(x)
    ef8 = jnp.zeros((_E, 8), F32).at[:, :3].set(edge_feat)

    xs = _sc_gather(xp128, src)

    msg0, msg1 = _tc_mlp(ef8, xs, w0, bs[0], ws[1], bs[1], ws[2], bs[2],
                         ws[3], bs[3], w4, b4)

    sum0, sum1, deg0, deg1 = _sc_segsum(msg0, msg1, dst)

    gsrc, gld, meta = _sc_partition(src, dst)

    nnb = params["nn_bias"][None, :]
    wa, ca, wb = ecw[0]
    a, b, degx = _tc_head(sum0, sum1, deg0, deg1, nnb, wa, ca, wb)

    dl = 128
    for l in (1, 2, 3):
        m0, m1 = _sc_segmin(b, gsrc, gld, meta, dl)
        wa, ca, wb = ecw[l]
        a, b = _tc_step(a, m0, m1, degx, wa, ca, wb)
        dl = wa.shape[1]

    m0, m1 = _sc_segmin(b, gsrc, gld, meta, dl)
    h = _tc_final(a, m0, m1, degx, dl)
    return h[:_N]
